# Initial kernel scaffold; baseline (speedup 1.0000x reference)
import functools

import jax
import jax.numpy as jnp
from jax import lax
from jax.experimental import pallas as pl
from jax.experimental.pallas import tpu as pltpu
from jax.experimental.pallas import tpu_sc as plsc

NC = 2
NS = 16
NW = NC * NS
CHUNK = 128
L2REG = 0.0005


def _mesh():
    return plsc.VectorSubcoreMesh(core_axis_name="c", subcore_axis_name="s")


def _make_deg_kernel(n, nch):
    rows_per = n // NS
    zb_rows = 125
    mesh = _mesh()

    @functools.partial(
        pl.kernel,
        mesh=mesh,
        out_type=(
            jax.ShapeDtypeStruct((NC, n, 16), jnp.float32),
            jax.ShapeDtypeStruct((NC, n, 16), jnp.float32),
        ),
        scratch_types=[
            pltpu.VMEM((CHUNK,), jnp.int32),
            pltpu.VMEM((CHUNK,), jnp.int32),
            pltpu.VMEM((CHUNK,), jnp.float32),
            pltpu.VMEM((CHUNK, 16), jnp.float32),
            pltpu.VMEM((CHUNK, 16), jnp.float32),
            pltpu.VMEM((125, 16), jnp.float32),
            pltpu.VMEM_SHARED((n, 16), jnp.float32),
            pltpu.VMEM_SHARED((n, 16), jnp.float32),
        ],
    )
    def k(src_h, dst_h, ew_h, out1, out2, src_v, dst_v, ew_v, r1, r2, zb, acc1, acc2):
        cid = lax.axis_index("c")
        sid = lax.axis_index("s")
        wid = cid * NS + sid

        def zfill(i, carry):
            zb[i, :] = jnp.zeros((16,), jnp.float32)
            return carry

        lax.fori_loop(0, zb_rows, zfill, 0)
        for t in range(rows_per // zb_rows):
            off = sid * rows_per + t * zb_rows
            pltpu.sync_copy(zb, acc1.at[pl.ds(off, zb_rows)])
            pltpu.sync_copy(zb, acc2.at[pl.ds(off, zb_rows)])
        plsc.subcore_barrier()

        def chunk(j, carry):
            pltpu.sync_copy(src_h.at[wid, j], src_v)
            pltpu.sync_copy(dst_h.at[wid, j], dst_v)
            pltpu.sync_copy(ew_h.at[wid, j], ew_v)

            def ebody(e, c2):
                w = ew_v[e]
                s = src_v[e]
                d = dst_v[e]
                r1[e, :] = jnp.full((16,), w, jnp.float32)
                m = jnp.where(s != d, jnp.float32(1.0), jnp.float32(0.0))
                r2[e, :] = jnp.full((16,), m, jnp.float32)
                return c2

            lax.fori_loop(0, CHUNK, ebody, 0)
            pltpu.sync_copy(r1, acc1.at[dst_v], add=True)
            pltpu.sync_copy(r2, acc2.at[dst_v], add=True)
            return carry

        lax.fori_loop(0, nch, chunk, 0)
        plsc.subcore_barrier()
        off = sid * rows_per
        pltpu.sync_copy(acc1.at[pl.ds(off, rows_per)], out1.at[cid, pl.ds(off, rows_per)])
        pltpu.sync_copy(acc2.at[pl.ds(off, rows_per)], out2.at[cid, pl.ds(off, rows_per)])

    return k


def _make_norm_kernel(nch):
    mesh = _mesh()

    @functools.partial(
        pl.kernel,
        mesh=mesh,
        out_type=(
            jax.ShapeDtypeStruct((NW, nch, CHUNK), jnp.float32),
            jax.ShapeDtypeStruct((NW, nch, CHUNK), jnp.float32),
        ),
        scratch_types=[
            pltpu.VMEM((CHUNK,), jnp.int32),
            pltpu.VMEM((CHUNK,), jnp.int32),
            pltpu.VMEM((CHUNK,), jnp.float32),
            pltpu.VMEM((CHUNK,), jnp.float32),
            pltpu.VMEM((CHUNK,), jnp.float32),
            pltpu.VMEM((CHUNK,), jnp.float32),
            pltpu.VMEM((CHUNK,), jnp.float32),
            pltpu.VMEM((CHUNK,), jnp.float32),
            pltpu.SemaphoreType.DMA,
        ],
    )
    def k(src_h, dst_h, ew_h, dinv_h, dinv2_h, norm_o, ew2_o,
          src_v, dst_v, ew_v, dis, did, di2, norm_v, ew2_v, sem):
        cid = lax.axis_index("c")
        sid = lax.axis_index("s")
        wid = cid * NS + sid

        def chunk(j, carry):
            pltpu.sync_copy(src_h.at[wid, j], src_v)
            pltpu.sync_copy(dst_h.at[wid, j], dst_v)
            pltpu.sync_copy(ew_h.at[wid, j], ew_v)
            pltpu.async_copy(dinv_h.at[src_v], dis, sem).wait()
            pltpu.async_copy(dinv_h.at[dst_v], did, sem).wait()
            pltpu.async_copy(dinv2_h.at[dst_v], di2, sem).wait()
            for t in range(CHUNK // 16):
                sl = pl.ds(t * 16, 16)
                norm_v[sl] = dis[sl] * ew_v[sl] * did[sl]
                m = jnp.where(src_v[sl] != dst_v[sl],
                              jnp.full((16,), 1.0, jnp.float32),
                              jnp.zeros((16,), jnp.float32))
                ew2_v[sl] = di2[sl] * m
            pltpu.sync_copy(norm_v, norm_o.at[wid, j])
            pltpu.sync_copy(ew2_v, ew2_o.at[wid, j])
            return carry

        lax.fori_loop(0, nch, chunk, 0)

    return k


def _make_spmm_kernel(n, nch, f):
    rows_per = n // NS
    zb_rows = 125
    nf = f // 16
    mesh = _mesh()

    @functools.partial(
        pl.kernel,
        mesh=mesh,
        out_type=jax.ShapeDtypeStruct((NC, n, f), jnp.float32),
        scratch_types=[
            pltpu.VMEM((CHUNK,), jnp.int32),
            pltpu.VMEM((CHUNK,), jnp.int32),
            pltpu.VMEM((CHUNK,), jnp.float32),
            pltpu.VMEM((CHUNK, f), jnp.float32),
            pltpu.VMEM((125, f), jnp.float32),
            pltpu.VMEM_SHARED((n, f), jnp.float32),
            pltpu.SemaphoreType.DMA,
        ],
    )
    def k(feat_h, src_h, dst_h, w_h, out, src_v, dst_v, w_v, rows, zb, acc, sem):
        cid = lax.axis_index("c")
        sid = lax.axis_index("s")
        wid = cid * NS + sid

        def zfill(i, carry):
            for q in range(nf):
                zb[i, pl.ds(q * 16, 16)] = jnp.zeros((16,), jnp.float32)
            return carry

        lax.fori_loop(0, zb_rows, zfill, 0)
        for t in range(rows_per // zb_rows):
            off = sid * rows_per + t * zb_rows
            pltpu.sync_copy(zb, acc.at[pl.ds(off, zb_rows)])
        plsc.subcore_barrier()

        def chunk(j, carry):
            pltpu.sync_copy(src_h.at[wid, j], src_v)
            pltpu.sync_copy(dst_h.at[wid, j], dst_v)
            pltpu.sync_copy(w_h.at[wid, j], w_v)
            pltpu.async_copy(feat_h.at[src_v], rows, sem).wait()

            def ebody(e, c2):
                w = w_v[e]
                for q in range(nf):
                    sl = pl.ds(q * 16, 16)
                    rows[e, sl] = rows[e, sl] * w
                return c2

            lax.fori_loop(0, CHUNK, ebody, 0)
            pltpu.sync_copy(rows, acc.at[dst_v], add=True)
            return carry

        lax.fori_loop(0, nch, chunk, 0)
        plsc.subcore_barrier()
        off = sid * rows_per
        pltpu.sync_copy(acc.at[pl.ds(off, rows_per)], out.at[cid, pl.ds(off, rows_per)])

    return k


def _dinv_body(d1, d2, dinv_o, dsq_o, dinv2_o):
    a = d1[0, :, :] + d1[1, :, :] + 1.0
    dinv = 1.0 / jnp.sqrt(a)
    dinv_o[...] = dinv[:, :1]
    dsq_o[...] = (dinv * dinv)[:, :1]
    b = jnp.maximum(d2[0, :, :] + d2[1, :, :] + 1.0, 1.0)
    dinv2_o[...] = (1.0 / b)[:, :1]


def _dinv_kernel(n, d1, d2):
    return pl.pallas_call(
        _dinv_body,
        out_shape=(
            jax.ShapeDtypeStruct((n, 1), jnp.float32),
            jax.ShapeDtypeStruct((n, 1), jnp.float32),
            jax.ShapeDtypeStruct((n, 1), jnp.float32),
        ),
    )(d1, d2)


def _matmul0_body(x, w, o):
    o[...] = jnp.dot(x[...], w[...], preferred_element_type=jnp.float32)


def _matmul0(x, w, bn):
    n, h = x.shape
    return pl.pallas_call(
        _matmul0_body,
        grid=(n // bn,),
        in_specs=[
            pl.BlockSpec((bn, h), lambda i: (i, 0)),
            pl.BlockSpec((h, w.shape[1]), lambda i: (0, 0)),
        ],
        out_specs=pl.BlockSpec((bn, w.shape[1]), lambda i: (i, 0)),
        out_shape=jax.ShapeDtypeStruct((n, w.shape[1]), jnp.float32),
    )(x, w)


def _combine_body(sc0, sc1, xw, dsq, b, jk, w, xwn_o, jk_o):
    h = sc0[...] + sc1[...] + dsq[...] * xw[...] + b[...]
    h = jnp.maximum(h, 0.0)
    jk_o[...] = jnp.maximum(jk[...], h)
    xwn_o[...] = jnp.dot(h, w[...], preferred_element_type=jnp.float32)


def _combine(sc0, sc1, xw, dsq, b, jk, w, bn):
    n, h = xw.shape
    ho = w.shape[1]
    spec = pl.BlockSpec((bn, h), lambda i: (i, 0))
    return pl.pallas_call(
        _combine_body,
        grid=(n // bn,),
        in_specs=[
            spec, spec, spec,
            pl.BlockSpec((bn, 1), lambda i: (i, 0)),
            pl.BlockSpec((1, h), lambda i: (0, 0)),
            spec,
            pl.BlockSpec((h, ho), lambda i: (0, 0)),
        ],
        out_specs=(pl.BlockSpec((bn, ho), lambda i: (i, 0)), spec),
        out_shape=(
            jax.ShapeDtypeStruct((n, ho), jnp.float32),
            jax.ShapeDtypeStruct((n, h), jnp.float32),
        ),
    )(sc0, sc1, xw, dsq, b, jk, w)


def _cluster_body(sc0, sc1, jk, dinv2, wout, wroot, bout, o):
    agg = sc0[...] + sc1[...] + dinv2[...] * jk[...]
    o[...] = (jnp.dot(agg, wout[...], preferred_element_type=jnp.float32)
              + jnp.dot(jk[...], wroot[...], preferred_element_type=jnp.float32)
              + bout[...])


def _cluster(sc0, sc1, jk, dinv2, wout, wroot, bout, bn):
    n, h = jk.shape
    spec = pl.BlockSpec((bn, h), lambda i: (i, 0))
    return pl.pallas_call(
        _cluster_body,
        grid=(n // bn,),
        in_specs=[
            spec, spec, spec,
            pl.BlockSpec((bn, 1), lambda i: (i, 0)),
            pl.BlockSpec((h, 1), lambda i: (0, 0)),
            pl.BlockSpec((h, 1), lambda i: (0, 0)),
            pl.BlockSpec((1, 1), lambda i: (0, 0)),
        ],
        out_specs=pl.BlockSpec((bn, 1), lambda i: (i, 0)),
        out_shape=jax.ShapeDtypeStruct((n, 1), jnp.float32),
    )(sc0, sc1, jk, dinv2, wout, wroot, bout)


def _l2_body(p, o):
    x = p[...]
    ss = jnp.sum(x * x, axis=2)
    ssum = jnp.sum(ss, axis=1, keepdims=True)
    o[...] = jnp.sum(jnp.sqrt(ssum), keepdims=True).reshape(1, 1) * L2REG


def _l2_kernel(p):
    return pl.pallas_call(
        _l2_body,
        out_shape=jax.ShapeDtypeStruct((1, 1), jnp.float32),
    )(p)


def kernel(x, edge_index, edge_weight, conv_weights, conv_biases,
           fc_out_W, fc_out_b, fc_root_W):
    n, f_in = x.shape
    h = conv_weights[0].shape[1]
    e = edge_weight.shape[0]
    nlayers = len(conv_weights)
    bn = 1000

    nch = -(-e // (NW * CHUNK))
    epad = NW * nch * CHUNK
    pad = epad - e
    src = edge_index[0]
    dst = edge_index[1]
    if pad:
        zi = jnp.zeros((pad,), jnp.int32)
        src = jnp.concatenate([src, zi])
        dst = jnp.concatenate([dst, zi])
        edge_weight = jnp.concatenate([edge_weight, jnp.zeros((pad,), jnp.float32)])
    src3 = src.reshape(NW, nch, CHUNK)
    dst3 = dst.reshape(NW, nch, CHUNK)
    ew3 = edge_weight.reshape(NW, nch, CHUNK)

    deg_k = _make_deg_kernel(n, nch)
    d1, d2 = deg_k(src3, dst3, ew3)
    dinv_col, dsq_col, dinv2_col = _dinv_kernel(n, d1, d2)

    norm_k = _make_norm_kernel(nch)
    norm3, ew23 = norm_k(src3, dst3, ew3, dinv_col.reshape(n), dinv2_col.reshape(n))

    spmm_k = _make_spmm_kernel(n, nch, h)

    eye = jnp.eye(h, dtype=jnp.float32)
    jk = jnp.zeros((n, h), jnp.float32)
    xw = _matmul0(x, conv_weights[0], bn)
    for i in range(nlayers):
        s = spmm_k(xw, src3, dst3, norm3)
        w_next = conv_weights[i + 1] if i + 1 < nlayers else eye
        b = conv_biases[i].reshape(1, h)
        xw, jk = _combine(s[0], s[1], xw, dsq_col, b, jk, w_next, bn)
    features = xw

    scc = spmm_k(jk, src3, dst3, ew23)
    out_col = _cluster(scc[0], scc[1], jk, dinv2_col, fc_out_W, fc_root_W,
                       fc_out_b.reshape(1, 1), bn)

    wstack = jnp.stack(conv_weights)
    bstack = jnp.stack(conv_biases)
    p = jnp.zeros((nlayers * 2 + 3, h, h), jnp.float32)
    p = p.at[0:nlayers].set(wstack)
    p = p.at[nlayers:2 * nlayers, 0, :].set(bstack)
    p = p.at[2 * nlayers, 0, :].set(fc_out_W[:, 0])
    p = p.at[2 * nlayers + 1, 0, :].set(fc_root_W[:, 0])
    p = p.at[2 * nlayers + 2, 0, 0].set(fc_out_b[0])
    l2 = _l2_kernel(p)

    return (out_col.reshape(-1), features, l2.reshape(()))


# speedup vs baseline: 4.1958x; 4.1958x over previous
import functools

import jax
import jax.numpy as jnp
from jax import lax
from jax.experimental import pallas as pl
from jax.experimental.pallas import tpu as pltpu
from jax.experimental.pallas import tpu_sc as plsc

NC = 2
NS = 16
NW = NC * NS
CHUNK = 128
L2REG = 0.0005


def _mesh():
    return plsc.VectorSubcoreMesh(core_axis_name="c", subcore_axis_name="s")


def _make_norm_kernel(nch):
    mesh = _mesh()

    @functools.partial(
        pl.kernel,
        mesh=mesh,
        out_type=(
            jax.ShapeDtypeStruct((NW, nch, CHUNK), jnp.float32),
            jax.ShapeDtypeStruct((NW, nch, CHUNK), jnp.float32),
        ),
        scratch_types=[
            pltpu.VMEM((CHUNK,), jnp.int32),
            pltpu.VMEM((CHUNK,), jnp.int32),
            pltpu.VMEM((CHUNK,), jnp.float32),
            pltpu.VMEM((CHUNK,), jnp.float32),
            pltpu.VMEM((CHUNK,), jnp.float32),
            pltpu.VMEM((CHUNK,), jnp.float32),
            pltpu.VMEM((CHUNK,), jnp.float32),
            pltpu.VMEM((CHUNK,), jnp.float32),
            pltpu.SemaphoreType.DMA,
        ],
    )
    def k(src_h, dst_h, ew_h, dinv_h, dinv2_h, norm_o, ew2_o,
          src_v, dst_v, ew_v, dis, did, di2, norm_v, ew2_v, sem):
        cid = lax.axis_index("c")
        sid = lax.axis_index("s")
        wid = cid * NS + sid

        def chunk(j, carry):
            pltpu.sync_copy(src_h.at[wid, j], src_v)
            pltpu.sync_copy(dst_h.at[wid, j], dst_v)
            pltpu.sync_copy(ew_h.at[wid, j], ew_v)
            pltpu.async_copy(dinv_h.at[src_v], dis, sem).wait()
            pltpu.async_copy(dinv_h.at[dst_v], did, sem).wait()
            pltpu.async_copy(dinv2_h.at[dst_v], di2, sem).wait()
            for t in range(CHUNK // 16):
                sl = pl.ds(t * 16, 16)
                norm_v[sl] = dis[sl] * ew_v[sl] * did[sl]
                m = jnp.where(src_v[sl] != dst_v[sl],
                              jnp.full((16,), 1.0, jnp.float32),
                              jnp.zeros((16,), jnp.float32))
                ew2_v[sl] = di2[sl] * m
            pltpu.sync_copy(norm_v, norm_o.at[wid, j])
            pltpu.sync_copy(ew2_v, ew2_o.at[wid, j])
            return carry

        lax.fori_loop(0, nch, chunk, 0)

    return k


def _make_spmm_kernel(npad, nch, f):
    rows_per = npad // NS
    zb_rows = rows_per // 5
    nf = f // 16
    mesh = _mesh()

    @functools.partial(
        pl.kernel,
        mesh=mesh,
        out_type=jax.ShapeDtypeStruct((NC, npad, f), jnp.float32),
        scratch_types=[
            pltpu.VMEM((CHUNK,), jnp.int32),
            pltpu.VMEM((CHUNK,), jnp.int32),
            pltpu.VMEM((CHUNK,), jnp.float32),
            pltpu.VMEM((CHUNK, f), jnp.float32),
            pltpu.VMEM((npad // NS // 5, f), jnp.float32),
            pltpu.VMEM_SHARED((npad, f), jnp.float32),
            pltpu.SemaphoreType.DMA,
        ],
    )
    def k(feat_h, src_h, dst_h, w_h, out, src_v, dst_v, w_v, rows, zb, acc, sem):
        cid = lax.axis_index("c")
        sid = lax.axis_index("s")
        wid = cid * NS + sid

        def zfill(i, carry):
            for q in range(nf):
                zb[i, pl.ds(q * 16, 16)] = jnp.zeros((16,), jnp.float32)
            return carry

        lax.fori_loop(0, zb_rows, zfill, 0)
        for t in range(rows_per // zb_rows):
            off = sid * rows_per + t * zb_rows
            pltpu.sync_copy(zb, acc.at[pl.ds(off, zb_rows)])
        plsc.subcore_barrier()

        def chunk(j, carry):
            pltpu.sync_copy(src_h.at[wid, j], src_v)
            pltpu.sync_copy(dst_h.at[wid, j], dst_v)
            pltpu.sync_copy(w_h.at[wid, j], w_v)
            pltpu.async_copy(feat_h.at[src_v], rows, sem).wait()

            def gbody(g, c2):
                off = pl.multiple_of(g * 16, 16)
                w16 = w_v[pl.ds(off, 16)]
                for jj in range(16):
                    e = off + jj
                    wsp = jnp.full((16,), w16[jj], jnp.float32)
                    for q in range(nf):
                        sl = pl.ds(q * 16, 16)
                        rows[e, sl] = rows[e, sl] * wsp
                return c2

            lax.fori_loop(0, CHUNK // 16, gbody, 0)
            pltpu.sync_copy(rows, acc.at[dst_v], add=True)
            return carry

        lax.fori_loop(0, nch, chunk, 0)
        plsc.subcore_barrier()
        off = sid * rows_per
        pltpu.sync_copy(acc.at[pl.ds(off, rows_per)], out.at[cid, pl.ds(off, rows_per)])

    return k


def _dinv_body(d1, d2, dinv_o, dsq_o, dinv2_o):
    a = d1[0, :, :1] + d1[1, :, :1] + 1.0
    dinv = 1.0 / jnp.sqrt(a)
    dinv_o[...] = dinv
    dsq_o[...] = dinv * dinv
    b = jnp.maximum(d2[0, :, :1] + d2[1, :, :1] + 1.0, 1.0)
    dinv2_o[...] = 1.0 / b


def _dinv_kernel(npad, d1, d2):
    bn = 1024
    spec = pl.BlockSpec((2, bn, 128), lambda i: (0, i, 0))
    ospec = pl.BlockSpec((bn, 1), lambda i: (i, 0))
    return pl.pallas_call(
        _dinv_body,
        grid=(npad // bn,),
        in_specs=[spec, spec],
        out_specs=(ospec, ospec, ospec),
        out_shape=(
            jax.ShapeDtypeStruct((npad, 1), jnp.float32),
            jax.ShapeDtypeStruct((npad, 1), jnp.float32),
            jax.ShapeDtypeStruct((npad, 1), jnp.float32),
        ),
    )(d1, d2)


def _matmul0_body(x, w, o):
    o[...] = jnp.dot(x[...], w[...], preferred_element_type=jnp.float32)


def _matmul0(x, w, bn):
    n, h = x.shape
    return pl.pallas_call(
        _matmul0_body,
        grid=(n // bn,),
        in_specs=[
            pl.BlockSpec((bn, h), lambda i: (i, 0)),
            pl.BlockSpec((h, w.shape[1]), lambda i: (0, 0)),
        ],
        out_specs=pl.BlockSpec((bn, w.shape[1]), lambda i: (i, 0)),
        out_shape=jax.ShapeDtypeStruct((n, w.shape[1]), jnp.float32),
    )(x, w)


def _combine_body(sc0, sc1, xw, dsq, b, jk, w, xwn_o, jk_o):
    h = sc0[...] + sc1[...] + dsq[...] * xw[...] + b[...]
    h = jnp.maximum(h, 0.0)
    jk_o[...] = jnp.maximum(jk[...], h)
    xwn_o[...] = jnp.dot(h, w[...], preferred_element_type=jnp.float32)


def _combine(sc0, sc1, xw, dsq, b, jk, w, bn):
    n, h = xw.shape
    ho = w.shape[1]
    spec = pl.BlockSpec((bn, h), lambda i: (i, 0))
    return pl.pallas_call(
        _combine_body,
        grid=(n // bn,),
        in_specs=[
            spec, spec, spec,
            pl.BlockSpec((bn, 1), lambda i: (i, 0)),
            pl.BlockSpec((1, h), lambda i: (0, 0)),
            spec,
            pl.BlockSpec((h, ho), lambda i: (0, 0)),
        ],
        out_specs=(pl.BlockSpec((bn, ho), lambda i: (i, 0)), spec),
        out_shape=(
            jax.ShapeDtypeStruct((n, ho), jnp.float32),
            jax.ShapeDtypeStruct((n, h), jnp.float32),
        ),
    )(sc0, sc1, xw, dsq, b, jk, w)


def _cluster_body(sc0, sc1, jk, dinv2, wout, wroot, bout, o):
    agg = sc0[...] + sc1[...] + dinv2[...] * jk[...]
    o[...] = (jnp.dot(agg, wout[...], preferred_element_type=jnp.float32)
              + jnp.dot(jk[...], wroot[...], preferred_element_type=jnp.float32)
              + bout[...])


def _cluster(sc0, sc1, jk, dinv2, wout, wroot, bout, bn):
    n, h = jk.shape
    spec = pl.BlockSpec((bn, h), lambda i: (i, 0))
    return pl.pallas_call(
        _cluster_body,
        grid=(n // bn,),
        in_specs=[
            spec, spec, spec,
            pl.BlockSpec((bn, 1), lambda i: (i, 0)),
            pl.BlockSpec((h, 1), lambda i: (0, 0)),
            pl.BlockSpec((h, 1), lambda i: (0, 0)),
            pl.BlockSpec((1, 1), lambda i: (0, 0)),
        ],
        out_specs=pl.BlockSpec((bn, 1), lambda i: (i, 0)),
        out_shape=jax.ShapeDtypeStruct((n, 1), jnp.float32),
    )(sc0, sc1, jk, dinv2, wout, wroot, bout)


def _l2_body(p, o):
    x = p[...]
    ss = jnp.sum(x * x, axis=2)
    ssum = jnp.sum(ss, axis=1, keepdims=True)
    o[...] = jnp.sum(jnp.sqrt(ssum), keepdims=True).reshape(1, 1) * L2REG


def _l2_kernel(p):
    return pl.pallas_call(
        _l2_body,
        out_shape=jax.ShapeDtypeStruct((1, 1), jnp.float32),
    )(p)


def kernel(x, edge_index, edge_weight, conv_weights, conv_biases,
           fc_out_W, fc_out_b, fc_root_W):
    n, f_in = x.shape
    h = conv_weights[0].shape[1]
    e = edge_weight.shape[0]
    nlayers = len(conv_weights)
    bn = 1000

    npad = -(-n // (NS * 40)) * (NS * 40)

    nch = -(-e // (NW * CHUNK))
    epad = NW * nch * CHUNK
    pad = epad - e
    src = edge_index[0]
    dst = edge_index[1]
    if pad:
        zi = jnp.zeros((pad,), jnp.int32)
        src = jnp.concatenate([src, zi])
        dst = jnp.concatenate([dst, zi])
        edge_weight = jnp.concatenate([edge_weight, jnp.zeros((pad,), jnp.float32)])
    src3 = src.reshape(NW, nch, CHUNK)
    dst3 = dst.reshape(NW, nch, CHUNK)
    ew3 = edge_weight.reshape(NW, nch, CHUNK)

    spmm_k = _make_spmm_kernel(npad, nch, h)

    ones_tab = jnp.ones((n, h), jnp.float32)
    mask3 = (src3 != dst3).astype(jnp.float32)
    d1 = spmm_k(ones_tab, src3, dst3, ew3)
    d2 = spmm_k(ones_tab, src3, dst3, mask3)
    dinv_col, dsq_col, dinv2_col = _dinv_kernel(npad, d1, d2)

    norm_k = _make_norm_kernel(nch)
    norm3, ew23 = norm_k(src3, dst3, ew3, dinv_col.reshape(npad), dinv2_col.reshape(npad))

    eye = jnp.eye(h, dtype=jnp.float32)
    jk = jnp.zeros((n, h), jnp.float32)
    xw = _matmul0(x, conv_weights[0], bn)
    for i in range(nlayers):
        s = spmm_k(xw, src3, dst3, norm3)
        w_next = conv_weights[i + 1] if i + 1 < nlayers else eye
        b = conv_biases[i].reshape(1, h)
        xw, jk = _combine(s[0], s[1], xw, dsq_col, b, jk, w_next, bn)
    features = xw

    scc = spmm_k(jk, src3, dst3, ew23)
    out_col = _cluster(scc[0], scc[1], jk, dinv2_col, fc_out_W, fc_root_W,
                       fc_out_b.reshape(1, 1), bn)

    wstack = jnp.stack(conv_weights)
    bstack = jnp.stack(conv_biases)
    p = jnp.zeros((nlayers * 2 + 3, h, h), jnp.float32)
    p = p.at[0:nlayers].set(wstack)
    p = p.at[nlayers:2 * nlayers, 0, :].set(bstack)
    p = p.at[2 * nlayers, 0, :].set(fc_out_W[:, 0])
    p = p.at[2 * nlayers + 1, 0, :].set(fc_root_W[:, 0])
    p = p.at[2 * nlayers + 2, 0, 0].set(fc_out_b[0])
    l2 = _l2_kernel(p)

    return (out_col.reshape(-1), features, l2.reshape(()))


# speedup vs baseline: 4.5401x; 1.0821x over previous
import functools

import jax
import jax.numpy as jnp
from jax import lax
from jax.experimental import pallas as pl
from jax.experimental.pallas import tpu as pltpu
from jax.experimental.pallas import tpu_sc as plsc

NC = 2
NS = 16
NW = NC * NS
CHUNK = 128
L2REG = 0.0005


def _mesh():
    return plsc.VectorSubcoreMesh(core_axis_name="c", subcore_axis_name="s")


def _make_deg_kernel(npad, nch):
    rows_per = npad // NS
    zb_rows = 40
    mesh = _mesh()

    @functools.partial(
        pl.kernel,
        mesh=mesh,
        out_type=jax.ShapeDtypeStruct((NC, npad, 128), jnp.float32),
        scratch_types=[
            pltpu.VMEM((8, CHUNK), jnp.int32),
            pltpu.VMEM((CHUNK,), jnp.float32),
            pltpu.VMEM((CHUNK, 128), jnp.float32),
            pltpu.VMEM((40, 128), jnp.float32),
            pltpu.VMEM_SHARED((npad, 128), jnp.float32),
        ],
    )
    def k(pk_h, w_h, out, pkb, wv, rows, zb, acc):
        cid = lax.axis_index("c")
        sid = lax.axis_index("s")
        wid = cid * NS + sid

        def zfill(i, carry):
            for q in range(8):
                zb[i, pl.ds(q * 16, 16)] = jnp.zeros((16,), jnp.float32)
            return carry

        lax.fori_loop(0, zb_rows, zfill, 0)
        for t in range(rows_per // zb_rows):
            off = sid * rows_per + t * zb_rows
            pltpu.sync_copy(zb, acc.at[pl.ds(off, zb_rows)])
        plsc.subcore_barrier()

        def chunk(j, carry):
            pltpu.sync_copy(pk_h.at[wid, j], pkb)
            pltpu.sync_copy(w_h.at[wid, j], wv)

            def gbody(g, c2):
                off = pl.multiple_of(g * 16, 16)
                s16 = pkb[0, pl.ds(off, 16)]
                d16 = pkb[1, pl.ds(off, 16)]
                w16 = wv[pl.ds(off, 16)]
                m16 = jnp.where(s16 != d16,
                                jnp.full((16,), 1.0, jnp.float32),
                                jnp.zeros((16,), jnp.float32))
                for jj in range(16):
                    e = off + jj
                    wsp = jnp.full((16,), w16[jj], jnp.float32)
                    msp = jnp.full((16,), m16[jj], jnp.float32)
                    for q in range(4):
                        rows[e, pl.ds(q * 16, 16)] = wsp
                    for q in range(4, 8):
                        rows[e, pl.ds(q * 16, 16)] = msp
                return c2

            lax.fori_loop(0, CHUNK // 16, gbody, 0)
            pltpu.sync_copy(rows, acc.at[pkb.at[1]], add=True)
            return carry

        lax.fori_loop(0, nch, chunk, 0)
        plsc.subcore_barrier()
        off = sid * rows_per
        pltpu.sync_copy(acc.at[pl.ds(off, rows_per)], out.at[cid, pl.ds(off, rows_per)])

    return k


def _make_spmm_kernel(npad, nch, f, weighted):
    rows_per = npad // NS
    zb_rows = 40
    nf = f // 16
    mesh = _mesh()

    scratch = [
        pltpu.VMEM((8, CHUNK), jnp.int32),
        pltpu.VMEM((8, CHUNK), jnp.int32),
        pltpu.VMEM((CHUNK, f), jnp.float32),
        pltpu.VMEM((CHUNK, f), jnp.float32),
        pltpu.VMEM((zb_rows, f), jnp.float32),
        pltpu.VMEM_SHARED((npad, f), jnp.float32),
        pltpu.SemaphoreType.DMA,
        pltpu.SemaphoreType.DMA,
    ]
    if weighted:
        scratch = ([pltpu.VMEM((CHUNK,), jnp.float32),
                    pltpu.VMEM((CHUNK,), jnp.float32)] + scratch)

    def body(feat_h, pk_h, *rest):
        if weighted:
            (w_h, out, wv0, wv1, pk0, pk1, rows0, rows1, zb, acc,
             gsem0, gsem1) = rest
        else:
            (out, pk0, pk1, rows0, rows1, zb, acc, gsem0, gsem1) = rest
            w_h = wv0 = wv1 = None
        cid = lax.axis_index("c")
        sid = lax.axis_index("s")
        wid = cid * NS + sid

        def zfill(i, carry):
            for q in range(nf):
                zb[i, pl.ds(q * 16, 16)] = jnp.zeros((16,), jnp.float32)
            return carry

        lax.fori_loop(0, zb_rows, zfill, 0)
        for t in range(rows_per // zb_rows):
            off = sid * rows_per + t * zb_rows
            pltpu.sync_copy(zb, acc.at[pl.ds(off, zb_rows)])
        plsc.subcore_barrier()

        pltpu.sync_copy(pk_h.at[wid, 0], pk0)
        if weighted:
            pltpu.sync_copy(w_h.at[wid, 0], wv0)
        pltpu.async_copy(feat_h.at[pk0.at[0]], rows0, gsem0)

        def half(j, pk, wv, rows, gsem_this, pk_nxt, wv_nxt, rows_nxt, gsem_nxt):
            pltpu.make_async_copy(feat_h.at[pk.at[0]], rows, gsem_this).wait()

            @pl.when(j + 1 < nch)
            def _():
                pltpu.sync_copy(pk_h.at[wid, j + 1], pk_nxt)
                if weighted:
                    pltpu.sync_copy(w_h.at[wid, j + 1], wv_nxt)
                pltpu.async_copy(feat_h.at[pk_nxt.at[0]], rows_nxt, gsem_nxt)

            if weighted:
                def gbody(g, c2):
                    off = pl.multiple_of(g * 16, 16)
                    w16 = wv[pl.ds(off, 16)]
                    for jj in range(16):
                        e = off + jj
                        wsp = jnp.full((16,), w16[jj], jnp.float32)
                        for q in range(nf):
                            sl = pl.ds(q * 16, 16)
                            rows[e, sl] = rows[e, sl] * wsp
                    return c2

                lax.fori_loop(0, CHUNK // 16, gbody, 0)
            pltpu.sync_copy(rows, acc.at[pk.at[1]], add=True)

        def loop(jj, carry):
            j = jj * 2
            half(j, pk0, wv0, rows0, gsem0, pk1, wv1, rows1, gsem1)
            half(j + 1, pk1, wv1, rows1, gsem1, pk0, wv0, rows0, gsem0)
            return carry

        lax.fori_loop(0, nch // 2, loop, 0)
        plsc.subcore_barrier()
        off = sid * rows_per
        pltpu.sync_copy(acc.at[pl.ds(off, rows_per)], out.at[cid, pl.ds(off, rows_per)])

    return functools.partial(
        pl.kernel,
        mesh=mesh,
        out_type=jax.ShapeDtypeStruct((NC, npad, f), jnp.float32),
        scratch_types=scratch,
    )(body)


def _dinv_body(d, dinv_o, dinv2_o):
    a = d[0, :, :1] + d[1, :, :1] + 1.0
    dinv_o[...] = 1.0 / jnp.sqrt(a)
    b = jnp.maximum(d[0, :, 64:65] + d[1, :, 64:65] + 1.0, 1.0)
    dinv2_o[...] = 1.0 / b


def _dinv_kernel(npad, d):
    bn = 1024
    ospec = pl.BlockSpec((bn, 1), lambda i: (i, 0))
    return pl.pallas_call(
        _dinv_body,
        grid=(npad // bn,),
        in_specs=[pl.BlockSpec((2, bn, 128), lambda i: (0, i, 0))],
        out_specs=(ospec, ospec),
        out_shape=(
            jax.ShapeDtypeStruct((npad, 1), jnp.float32),
            jax.ShapeDtypeStruct((npad, 1), jnp.float32),
        ),
    )(d)


def _matmul0_body(x, w, dinv, o):
    o[...] = dinv[...] * jnp.dot(x[...], w[...],
                                 preferred_element_type=jnp.float32)


def _matmul0(x, w, dinv, bn):
    n, h = x.shape
    return pl.pallas_call(
        _matmul0_body,
        grid=(n // bn,),
        in_specs=[
            pl.BlockSpec((bn, h), lambda i: (i, 0)),
            pl.BlockSpec((h, w.shape[1]), lambda i: (0, 0)),
            pl.BlockSpec((bn, 1), lambda i: (i, 0)),
        ],
        out_specs=pl.BlockSpec((bn, w.shape[1]), lambda i: (i, 0)),
        out_shape=jax.ShapeDtypeStruct((n, w.shape[1]), jnp.float32),
    )(x, w, dinv)


def _combine_body(sc0, sc1, y, dinv, b, jk, w, y_o, jk_o, h_o):
    h = dinv[...] * (sc0[...] + sc1[...] + y[...]) + b[...]
    h = jnp.maximum(h, 0.0)
    h_o[...] = h
    jk_o[...] = jnp.maximum(jk[...], h)
    y_o[...] = dinv[...] * jnp.dot(h, w[...], preferred_element_type=jnp.float32)


def _combine(sc0, sc1, y, dinv, b, jk, w, bn):
    n, h = y.shape
    spec = pl.BlockSpec((bn, h), lambda i: (i, 0))
    return pl.pallas_call(
        _combine_body,
        grid=(n // bn,),
        in_specs=[
            spec, spec, spec,
            pl.BlockSpec((bn, 1), lambda i: (i, 0)),
            pl.BlockSpec((1, h), lambda i: (0, 0)),
            spec,
            pl.BlockSpec((h, h), lambda i: (0, 0)),
        ],
        out_specs=(spec, spec, spec),
        out_shape=(
            jax.ShapeDtypeStruct((n, h), jnp.float32),
            jax.ShapeDtypeStruct((n, h), jnp.float32),
            jax.ShapeDtypeStruct((n, h), jnp.float32),
        ),
    )(sc0, sc1, y, dinv, b, jk, w)


def _cluster_body(sc0, sc1, jk, dinv2, wout, wroot, bout, o):
    agg = dinv2[...] * (sc0[...] + sc1[...] + jk[...])
    o[...] = (jnp.dot(agg, wout[...], preferred_element_type=jnp.float32)
              + jnp.dot(jk[...], wroot[...], preferred_element_type=jnp.float32)
              + bout[...])


def _cluster(sc0, sc1, jk, dinv2, wout, wroot, bout, bn):
    n, h = jk.shape
    spec = pl.BlockSpec((bn, h), lambda i: (i, 0))
    return pl.pallas_call(
        _cluster_body,
        grid=(n // bn,),
        in_specs=[
            spec, spec, spec,
            pl.BlockSpec((bn, 1), lambda i: (i, 0)),
            pl.BlockSpec((h, 1), lambda i: (0, 0)),
            pl.BlockSpec((h, 1), lambda i: (0, 0)),
            pl.BlockSpec((1, 1), lambda i: (0, 0)),
        ],
        out_specs=pl.BlockSpec((bn, 1), lambda i: (i, 0)),
        out_shape=jax.ShapeDtypeStruct((n, 1), jnp.float32),
    )(sc0, sc1, jk, dinv2, wout, wroot, bout)


def _l2_body(p, o):
    x = p[...]
    ss = jnp.sum(x * x, axis=2)
    ssum = jnp.sum(ss, axis=1, keepdims=True)
    o[...] = jnp.sum(jnp.sqrt(ssum), keepdims=True).reshape(1, 1) * L2REG


def _l2_kernel(p):
    return pl.pallas_call(
        _l2_body,
        out_shape=jax.ShapeDtypeStruct((1, 1), jnp.float32),
    )(p)


def kernel(x, edge_index, edge_weight, conv_weights, conv_biases,
           fc_out_W, fc_out_b, fc_root_W):
    n, f_in = x.shape
    h = conv_weights[0].shape[1]
    e = edge_weight.shape[0]
    nlayers = len(conv_weights)
    bn = 1000

    npad = -(-n // (NS * 40)) * (NS * 40)

    nch = -(-e // (NW * CHUNK))
    nch = nch + (nch % 2)
    epad = NW * nch * CHUNK
    pad = epad - e
    src = edge_index[0]
    dst = edge_index[1]
    if pad:
        zi = jnp.zeros((pad,), jnp.int32)
        src = jnp.concatenate([src, zi])
        dst = jnp.concatenate([dst, zi])
        edge_weight = jnp.concatenate([edge_weight, jnp.zeros((pad,), jnp.float32)])
    src3 = src.reshape(NW, nch, CHUNK)
    dst3 = dst.reshape(NW, nch, CHUNK)
    ew3 = edge_weight.reshape(NW, nch, CHUNK)
    srcc3 = jnp.where(src3 == dst3, jnp.int32(n), src3)
    zpad = jnp.zeros((NW, nch, 6, CHUNK), jnp.int32)
    pk = jnp.concatenate(
        [jnp.stack([src3, dst3], axis=2), zpad], axis=2)
    pkc = jnp.concatenate(
        [jnp.stack([srcc3, dst3], axis=2), zpad], axis=2)

    deg_k = _make_deg_kernel(npad, nch)
    d12 = deg_k(pk, ew3)
    dinv_col, dinv2_col = _dinv_kernel(npad, d12)

    spmm_w = _make_spmm_kernel(npad, nch, h, True)
    spmm_u = _make_spmm_kernel(npad, nch, h, False)

    eye = jnp.eye(h, dtype=jnp.float32)
    jk = jnp.zeros((n, h), jnp.float32)
    feats = None
    y = _matmul0(x, conv_weights[0], dinv_col, bn)
    for i in range(nlayers):
        s = spmm_w(y, pk, ew3)
        w_next = conv_weights[i + 1] if i + 1 < nlayers else eye
        b = conv_biases[i].reshape(1, h)
        y, jk, feats = _combine(s[0], s[1], y, dinv_col, b, jk, w_next, bn)
    features = feats

    jk_ext = jnp.zeros((n + 8, h), jnp.float32).at[:n].set(jk)
    scc = spmm_u(jk_ext, pkc)
    out_col = _cluster(scc[0], scc[1], jk, dinv2_col, fc_out_W, fc_root_W,
                       fc_out_b.reshape(1, 1), bn)

    wstack = jnp.stack(conv_weights)
    bstack = jnp.stack(conv_biases)
    p = jnp.zeros((nlayers * 2 + 3, h, h), jnp.float32)
    p = p.at[0:nlayers].set(wstack)
    p = p.at[nlayers:2 * nlayers, 0, :].set(bstack)
    p = p.at[2 * nlayers, 0, :].set(fc_out_W[:, 0])
    p = p.at[2 * nlayers + 1, 0, :].set(fc_root_W[:, 0])
    p = p.at[2 * nlayers + 2, 0, 0].set(fc_out_b[0])
    l2 = _l2_kernel(p)

    return (out_col.reshape(-1), features, l2.reshape(()))


# speedup vs baseline: 5.0554x; 1.1135x over previous
import functools

import jax
import jax.numpy as jnp
from jax import lax
from jax.experimental import pallas as pl
from jax.experimental.pallas import tpu as pltpu
from jax.experimental.pallas import tpu_sc as plsc

NC = 2
NS = 16
NW = NC * NS
CHUNK = 128
L2REG = 0.0005


def _mesh():
    return plsc.VectorSubcoreMesh(core_axis_name="c", subcore_axis_name="s")


def _make_deg_kernel(npad, nch):
    rows_per = npad // NS
    zb_rows = 40
    mesh = _mesh()

    @functools.partial(
        pl.kernel,
        mesh=mesh,
        out_type=jax.ShapeDtypeStruct((NC, npad, 128), jnp.float32),
        scratch_types=[
            pltpu.VMEM((8, CHUNK), jnp.int32),
            pltpu.VMEM((8, CHUNK), jnp.int32),
            pltpu.VMEM((CHUNK,), jnp.float32),
            pltpu.VMEM((CHUNK,), jnp.float32),
            pltpu.VMEM((CHUNK, 128), jnp.float32),
            pltpu.VMEM((CHUNK, 128), jnp.float32),
            pltpu.VMEM((40, 128), jnp.float32),
            pltpu.VMEM_SHARED((npad, 128), jnp.float32),
            pltpu.SemaphoreType.DMA,
            pltpu.SemaphoreType.DMA,
            pltpu.SemaphoreType.DMA,
            pltpu.SemaphoreType.DMA,
            pltpu.SemaphoreType.DMA,
            pltpu.SemaphoreType.DMA,
        ],
    )
    def k(pk_h, w_h, out, pk0, pk1, wv0, wv1, rows0, rows1, zb, acc,
          psem0, psem1, qsem0, qsem1, ssem0, ssem1):
        cid = lax.axis_index("c")
        sid = lax.axis_index("s")
        wid = cid * NS + sid

        def zfill(i, carry):
            for q in range(8):
                zb[i, pl.ds(q * 16, 16)] = jnp.zeros((16,), jnp.float32)
            return carry

        lax.fori_loop(0, zb_rows, zfill, 0)
        for t in range(rows_per // zb_rows):
            off = sid * rows_per + t * zb_rows
            pltpu.sync_copy(zb, acc.at[pl.ds(off, zb_rows)])
        plsc.subcore_barrier()

        pltpu.sync_copy(pk_h.at[wid, 0], pk0)
        pltpu.sync_copy(w_h.at[wid, 0], wv0)

        def half(j, pk, wv, rows, ssem, pk_nxt, wv_nxt, rows_nxt,
                 psem_nxt, qsem_nxt, ssem_nxt):
            @pl.when(j + 1 < nch)
            def _():
                pltpu.async_copy(pk_h.at[wid, j + 1], pk_nxt, psem_nxt)
                pltpu.async_copy(w_h.at[wid, j + 1], wv_nxt, qsem_nxt)

            def gbody(g, c2):
                off = pl.multiple_of(g * 16, 16)
                s16 = pk[0, pl.ds(off, 16)]
                d16 = pk[1, pl.ds(off, 16)]
                w16 = wv[pl.ds(off, 16)]
                m16 = jnp.where(s16 != d16,
                                jnp.full((16,), 1.0, jnp.float32),
                                jnp.zeros((16,), jnp.float32))
                for jj in range(16):
                    e = off + jj
                    wsp = jnp.full((16,), w16[jj], jnp.float32)
                    msp = jnp.full((16,), m16[jj], jnp.float32)
                    for q in range(4):
                        rows[e, pl.ds(q * 16, 16)] = wsp
                    for q in range(4, 8):
                        rows[e, pl.ds(q * 16, 16)] = msp
                return c2

            lax.fori_loop(0, CHUNK // 16, gbody, 0)

            @pl.when(j + 1 < nch)
            def _():
                pltpu.make_async_copy(pk_h.at[wid, j + 1], pk_nxt, psem_nxt).wait()
                pltpu.make_async_copy(w_h.at[wid, j + 1], wv_nxt, qsem_nxt).wait()

            pltpu.sync_copy(rows, acc.at[pk.at[1]], add=True)

        def loop(jj, carry):
            j = jj * 2
            half(j, pk0, wv0, rows0, ssem0, pk1, wv1, rows1, psem1, qsem1, ssem1)
            half(j + 1, pk1, wv1, rows1, ssem1, pk0, wv0, rows0, psem0, qsem0, ssem0)
            return carry

        lax.fori_loop(0, nch // 2, loop, 0)
        plsc.subcore_barrier()
        off = sid * rows_per
        pltpu.sync_copy(acc.at[pl.ds(off, rows_per)], out.at[cid, pl.ds(off, rows_per)])

    return k


def _make_spmm_kernel(npad, nch, f, weighted):
    rows_per = npad // NS
    zb_rows = 40
    nf = f // 16
    mesh = _mesh()

    scratch = [
        pltpu.VMEM((8, CHUNK), jnp.int32),
        pltpu.VMEM((8, CHUNK), jnp.int32),
        pltpu.VMEM((CHUNK, f), jnp.float32),
        pltpu.VMEM((CHUNK, f), jnp.float32),
        pltpu.VMEM((zb_rows, f), jnp.float32),
        pltpu.VMEM_SHARED((npad, f), jnp.float32),
        pltpu.SemaphoreType.DMA,
        pltpu.SemaphoreType.DMA,
        pltpu.SemaphoreType.DMA,
        pltpu.SemaphoreType.DMA,
        pltpu.SemaphoreType.DMA,
        pltpu.SemaphoreType.DMA,
    ]
    if weighted:
        scratch = ([pltpu.VMEM((CHUNK,), jnp.float32),
                    pltpu.VMEM((CHUNK,), jnp.float32)] + scratch
                   + [pltpu.SemaphoreType.DMA, pltpu.SemaphoreType.DMA])

    def body(feat_h, pk_h, *rest):
        if weighted:
            (w_h, out, wv0, wv1, pk0, pk1, rows0, rows1, zb, acc,
             gsem0, gsem1, psem0, psem1, ssem0, ssem1, qsem0, qsem1) = rest
        else:
            (out, pk0, pk1, rows0, rows1, zb, acc,
             gsem0, gsem1, psem0, psem1, ssem0, ssem1) = rest
            w_h = wv0 = wv1 = qsem0 = qsem1 = None
        cid = lax.axis_index("c")
        sid = lax.axis_index("s")
        wid = cid * NS + sid

        def zfill(i, carry):
            for q in range(nf):
                zb[i, pl.ds(q * 16, 16)] = jnp.zeros((16,), jnp.float32)
            return carry

        lax.fori_loop(0, zb_rows, zfill, 0)
        for t in range(rows_per // zb_rows):
            off = sid * rows_per + t * zb_rows
            pltpu.sync_copy(zb, acc.at[pl.ds(off, zb_rows)])
        plsc.subcore_barrier()

        pltpu.sync_copy(pk_h.at[wid, 0], pk0)
        if weighted:
            pltpu.sync_copy(w_h.at[wid, 0], wv0)
        pltpu.async_copy(feat_h.at[pk0.at[0]], rows0, gsem0)

        def half(j, pk, wv, rows, gsem, ssem, pk_nxt, wv_nxt, rows_nxt,
                 gsem_nxt, psem_nxt, qsem_nxt, ssem_nxt):
            @pl.when(j + 1 < nch)
            def _():
                pltpu.async_copy(pk_h.at[wid, j + 1], pk_nxt, psem_nxt)
                if weighted:
                    pltpu.async_copy(w_h.at[wid, j + 1], wv_nxt, qsem_nxt)

            pltpu.make_async_copy(feat_h.at[pk.at[0]], rows, gsem).wait()

            @pl.when(j + 1 < nch)
            def _():
                pltpu.make_async_copy(pk_h.at[wid, j + 1], pk_nxt, psem_nxt).wait()
                pltpu.async_copy(feat_h.at[pk_nxt.at[0]], rows_nxt, gsem_nxt)
                if weighted:
                    pltpu.make_async_copy(w_h.at[wid, j + 1], wv_nxt, qsem_nxt).wait()

            if weighted:
                def gbody(g, c2):
                    off = pl.multiple_of(g * 16, 16)
                    w16 = wv[pl.ds(off, 16)]
                    for jj in range(16):
                        e = off + jj
                        wsp = jnp.full((16,), w16[jj], jnp.float32)
                        for q in range(nf):
                            sl = pl.ds(q * 16, 16)
                            rows[e, sl] = rows[e, sl] * wsp
                    return c2

                lax.fori_loop(0, CHUNK // 16, gbody, 0)
            pltpu.sync_copy(rows, acc.at[pk.at[1]], add=True)

        def loop(jj, carry):
            j = jj * 2
            half(j, pk0, wv0, rows0, gsem0, ssem0,
                 pk1, wv1, rows1, gsem1, psem1, qsem1, ssem1)
            half(j + 1, pk1, wv1, rows1, gsem1, ssem1,
                 pk0, wv0, rows0, gsem0, psem0, qsem0, ssem0)
            return carry

        lax.fori_loop(0, nch // 2, loop, 0)
        plsc.subcore_barrier()
        off = sid * rows_per
        pltpu.sync_copy(acc.at[pl.ds(off, rows_per)], out.at[cid, pl.ds(off, rows_per)])

    return functools.partial(
        pl.kernel,
        mesh=mesh,
        out_type=jax.ShapeDtypeStruct((NC, npad, f), jnp.float32),
        scratch_types=scratch,
    )(body)


def _dinv_body(d, dinv_o, dinv2_o):
    a = d[0, :, :1] + d[1, :, :1] + 1.0
    dinv_o[...] = 1.0 / jnp.sqrt(a)
    b = jnp.maximum(d[0, :, 64:65] + d[1, :, 64:65] + 1.0, 1.0)
    dinv2_o[...] = 1.0 / b


def _dinv_kernel(npad, d):
    bn = 1024
    ospec = pl.BlockSpec((bn, 1), lambda i: (i, 0))
    return pl.pallas_call(
        _dinv_body,
        grid=(npad // bn,),
        in_specs=[pl.BlockSpec((2, bn, 128), lambda i: (0, i, 0))],
        out_specs=(ospec, ospec),
        out_shape=(
            jax.ShapeDtypeStruct((npad, 1), jnp.float32),
            jax.ShapeDtypeStruct((npad, 1), jnp.float32),
        ),
    )(d)


def _matmul0_body(x, w, dinv, o):
    o[...] = dinv[...] * jnp.dot(x[...], w[...],
                                 preferred_element_type=jnp.float32)


def _matmul0(x, w, dinv, bn):
    n, h = x.shape
    return pl.pallas_call(
        _matmul0_body,
        grid=(n // bn,),
        in_specs=[
            pl.BlockSpec((bn, h), lambda i: (i, 0)),
            pl.BlockSpec((h, w.shape[1]), lambda i: (0, 0)),
            pl.BlockSpec((bn, 1), lambda i: (i, 0)),
        ],
        out_specs=pl.BlockSpec((bn, w.shape[1]), lambda i: (i, 0)),
        out_shape=jax.ShapeDtypeStruct((n, w.shape[1]), jnp.float32),
    )(x, w, dinv)


def _combine_body(sc0, sc1, y, dinv, b, jk, w, y_o, jk_o, h_o):
    h = dinv[...] * (sc0[...] + sc1[...] + y[...]) + b[...]
    h = jnp.maximum(h, 0.0)
    h_o[...] = h
    jk_o[...] = jnp.maximum(jk[...], h)
    y_o[...] = dinv[...] * jnp.dot(h, w[...], preferred_element_type=jnp.float32)


def _combine(sc0, sc1, y, dinv, b, jk, w, bn):
    n, h = y.shape
    spec = pl.BlockSpec((bn, h), lambda i: (i, 0))
    return pl.pallas_call(
        _combine_body,
        grid=(n // bn,),
        in_specs=[
            spec, spec, spec,
            pl.BlockSpec((bn, 1), lambda i: (i, 0)),
            pl.BlockSpec((1, h), lambda i: (0, 0)),
            spec,
            pl.BlockSpec((h, h), lambda i: (0, 0)),
        ],
        out_specs=(spec, spec, spec),
        out_shape=(
            jax.ShapeDtypeStruct((n, h), jnp.float32),
            jax.ShapeDtypeStruct((n, h), jnp.float32),
            jax.ShapeDtypeStruct((n, h), jnp.float32),
        ),
    )(sc0, sc1, y, dinv, b, jk, w)


def _cluster_body(sc0, sc1, jk, dinv2, wout, wroot, bout, o):
    agg = dinv2[...] * (sc0[...] + sc1[...] + jk[...])
    o[...] = (jnp.dot(agg, wout[...], preferred_element_type=jnp.float32)
              + jnp.dot(jk[...], wroot[...], preferred_element_type=jnp.float32)
              + bout[...])


def _cluster(sc0, sc1, jk, dinv2, wout, wroot, bout, bn):
    n, h = jk.shape
    spec = pl.BlockSpec((bn, h), lambda i: (i, 0))
    return pl.pallas_call(
        _cluster_body,
        grid=(n // bn,),
        in_specs=[
            spec, spec, spec,
            pl.BlockSpec((bn, 1), lambda i: (i, 0)),
            pl.BlockSpec((h, 1), lambda i: (0, 0)),
            pl.BlockSpec((h, 1), lambda i: (0, 0)),
            pl.BlockSpec((1, 1), lambda i: (0, 0)),
        ],
        out_specs=pl.BlockSpec((bn, 1), lambda i: (i, 0)),
        out_shape=jax.ShapeDtypeStruct((n, 1), jnp.float32),
    )(sc0, sc1, jk, dinv2, wout, wroot, bout)


def _l2_body(p, o):
    x = p[...]
    ss = jnp.sum(x * x, axis=2)
    ssum = jnp.sum(ss, axis=1, keepdims=True)
    o[...] = jnp.sum(jnp.sqrt(ssum), keepdims=True).reshape(1, 1) * L2REG


def _l2_kernel(p):
    return pl.pallas_call(
        _l2_body,
        out_shape=jax.ShapeDtypeStruct((1, 1), jnp.float32),
    )(p)


def kernel(x, edge_index, edge_weight, conv_weights, conv_biases,
           fc_out_W, fc_out_b, fc_root_W):
    n, f_in = x.shape
    h = conv_weights[0].shape[1]
    e = edge_weight.shape[0]
    nlayers = len(conv_weights)
    bn = 1000

    npad = -(-n // (NS * 40)) * (NS * 40)

    nch = -(-e // (NW * CHUNK))
    nch = nch + (nch % 2)
    epad = NW * nch * CHUNK
    pad = epad - e
    src = edge_index[0]
    dst = edge_index[1]
    if pad:
        zi = jnp.zeros((pad,), jnp.int32)
        src = jnp.concatenate([src, zi])
        dst = jnp.concatenate([dst, zi])
        edge_weight = jnp.concatenate([edge_weight, jnp.zeros((pad,), jnp.float32)])
    src3 = src.reshape(NW, nch, CHUNK)
    dst3 = dst.reshape(NW, nch, CHUNK)
    ew3 = edge_weight.reshape(NW, nch, CHUNK)
    srcc3 = jnp.where(src3 == dst3, jnp.int32(n), src3)
    zpad = jnp.zeros((NW, nch, 6, CHUNK), jnp.int32)
    pk = jnp.concatenate(
        [jnp.stack([src3, dst3], axis=2), zpad], axis=2)
    pkc = jnp.concatenate(
        [jnp.stack([srcc3, dst3], axis=2), zpad], axis=2)

    deg_k = _make_deg_kernel(npad, nch)
    d12 = deg_k(pk, ew3)
    dinv_col, dinv2_col = _dinv_kernel(npad, d12)

    spmm_w = _make_spmm_kernel(npad, nch, h, True)
    spmm_u = _make_spmm_kernel(npad, nch, h, False)

    eye = jnp.eye(h, dtype=jnp.float32)
    jk = jnp.zeros((n, h), jnp.float32)
    feats = None
    y = _matmul0(x, conv_weights[0], dinv_col, bn)
    for i in range(nlayers):
        s = spmm_w(y, pk, ew3)
        w_next = conv_weights[i + 1] if i + 1 < nlayers else eye
        b = conv_biases[i].reshape(1, h)
        y, jk, feats = _combine(s[0], s[1], y, dinv_col, b, jk, w_next, bn)
    features = feats

    jk_ext = jnp.zeros((n + 8, h), jnp.float32).at[:n].set(jk)
    scc = spmm_u(jk_ext, pkc)
    out_col = _cluster(scc[0], scc[1], jk, dinv2_col, fc_out_W, fc_root_W,
                       fc_out_b.reshape(1, 1), bn)

    wstack = jnp.stack(conv_weights)
    bstack = jnp.stack(conv_biases)
    p = jnp.zeros((nlayers * 2 + 3, h, h), jnp.float32)
    p = p.at[0:nlayers].set(wstack)
    p = p.at[nlayers:2 * nlayers, 0, :].set(bstack)
    p = p.at[2 * nlayers, 0, :].set(fc_out_W[:, 0])
    p = p.at[2 * nlayers + 1, 0, :].set(fc_root_W[:, 0])
    p = p.at[2 * nlayers + 2, 0, 0].set(fc_out_b[0])
    l2 = _l2_kernel(p)

    return (out_col.reshape(-1), features, l2.reshape(()))
